# baseline (speedup 1.0000x reference)
import dataclasses
import functools

import jax
import jax.numpy as jnp
from jax import lax
from jax.experimental import pallas as pl
from jax.experimental.pallas import tpu as pltpu
from jax.experimental.pallas import tpu_sc as plsc

BATCH = 16384
DIM = 64
NC = 2
NS = 16
NW = NC * NS
BPW = BATCH // NW
GROUPS = BPW // 16


def _sc_body(hi_hbm, ri_hbm, ti_hbm, ent_hbm, rel_hbm, out_hbm,
             hi_v, ri_v, ti_v, hrows, rrows, trows, out_v, sem):
    wid = lax.axis_index("s") * NC + lax.axis_index("c")
    base = wid * BPW

    pltpu.sync_copy(hi_hbm.at[pl.ds(base, BPW)], hi_v)
    pltpu.sync_copy(ri_hbm.at[pl.ds(base, BPW)], ri_v)
    pltpu.sync_copy(ti_hbm.at[pl.ds(base, BPW)], ti_v)

    ch = pltpu.async_copy(ent_hbm.at[hi_v], hrows, sem)
    cr = pltpu.async_copy(rel_hbm.at[ri_v], rrows, sem)
    ct = pltpu.async_copy(ent_hbm.at[ti_v], trows, sem)
    ch.wait()
    cr.wait()
    ct.wait()

    lanes = lax.iota(jnp.int32, 16)

    @pl.loop(0, GROUPS)
    def _(g):
        rows16 = g * 16 + lanes
        acc = jnp.zeros((16,), jnp.float32)
        for d in range(DIM):
            col = jnp.full((16,), d, jnp.int32)
            h = plsc.load_gather(hrows, [rows16, col])
            r = plsc.load_gather(rrows, [rows16, col])
            t = plsc.load_gather(trows, [rows16, col])
            acc = acc + h * r * t
        out_v[pl.ds(g * 16, 16)] = acc

    pltpu.sync_copy(out_v, out_hbm.at[pl.ds(base, BPW)])


@jax.jit
def kernel(sample, entity_embedding, relation_embedding):
    hi = sample[:, 0].astype(jnp.int32)
    ri = sample[:, 1].astype(jnp.int32)
    ti = sample[:, 2].astype(jnp.int32)

    mesh = plsc.VectorSubcoreMesh(core_axis_name="c", subcore_axis_name="s")
    cp = pltpu.CompilerParams(use_tc_tiling_on_sc=False)
    if "needs_layout_passes" in pltpu.CompilerParams.__dataclass_fields__:
        cp = dataclasses.replace(cp, needs_layout_passes=False)
    run = pl.kernel(
        _sc_body,
        out_type=jax.ShapeDtypeStruct((BATCH,), jnp.float32),
        mesh=mesh,
        scratch_types=[
            pltpu.VMEM((BPW,), jnp.int32),
            pltpu.VMEM((BPW,), jnp.int32),
            pltpu.VMEM((BPW,), jnp.int32),
            pltpu.VMEM((BPW, DIM), jnp.float32),
            pltpu.VMEM((BPW, DIM), jnp.float32),
            pltpu.VMEM((BPW, DIM), jnp.float32),
            pltpu.VMEM((BPW,), jnp.float32),
            pltpu.SemaphoreType.DMA,
        ],
        compiler_params=cp,
    )
    score = run(hi, ri, ti, entity_embedding, relation_embedding)
    return score.reshape(BATCH, 1)
